# baseline (speedup 1.0000x reference)
import functools

import jax
import jax.numpy as jnp
from jax.experimental import pallas as pl

_HI = jax.lax.Precision.HIGHEST


def _mm_body(a_ref, x_ref, b_ref, o_ref, *, act):
    y = jax.lax.dot_general(
        a_ref[...].astype(jnp.bfloat16), x_ref[...].astype(jnp.bfloat16),
        (((1,), (0,)), ((), ())), preferred_element_type=jnp.float32)
    y = y + b_ref[...]
    if act == "lrelu":
        y = jnp.where(y > 0, y, 0.1 * y)
    elif act == "tanh":
        y = jnp.tanh(y)
    o_ref[...] = y


@functools.partial(jax.jit, static_argnames=("act", "nb"))
def _mm(a, x, bias, act="none", nb=2048):
    m, k = a.shape
    k2, n = x.shape
    assert k == k2
    nb = min(nb, n)
    grid = (n // nb,)
    return pl.pallas_call(
        functools.partial(_mm_body, act=act),
        grid=grid,
        in_specs=[
            pl.BlockSpec((m, k), lambda i: (0, 0)),
            pl.BlockSpec((k, nb), lambda i: (0, i)),
            pl.BlockSpec((m, 1), lambda i: (0, 0)),
        ],
        out_specs=pl.BlockSpec((m, nb), lambda i: (0, i)),
        out_shape=jax.ShapeDtypeStruct((m, n), jnp.float32),
    )(a, x, bias)


def _vq_body(emb_ref, cb_ref, lw_ref, lb_ref, fw_ref, fb_ref, o_ref,
             *, n_batch, tf):
    emb = emb_ref[...]
    cb = cb_ref[...]
    kk, c = cb.shape
    n = emb.shape[1]
    s = jax.lax.dot_general(cb.astype(jnp.bfloat16),
                            emb.astype(jnp.bfloat16), (((1,), (0,)), ((), ())),
                            preferred_element_type=jnp.float32)
    cn2 = jnp.sum(cb * cb, axis=1, keepdims=True)
    d = cn2 - 2.0 * s
    dmin = jnp.min(d, axis=0, keepdims=True)
    jidx = jax.lax.broadcasted_iota(jnp.int32, (kk, n), 0)
    codes = jnp.min(jnp.where(d == dmin, jidx, kk), axis=0, keepdims=True)
    onehot = (jidx == codes).astype(jnp.float32)
    emb_r = jax.lax.dot_general(cb, onehot, (((0,), (0,)), ((), ())),
                                precision=_HI,
                                preferred_element_type=jnp.float32)
    bidx = jax.lax.broadcasted_iota(jnp.int32, (n, n_batch), 0) // tf
    bcol = jax.lax.broadcasted_iota(jnp.int32, (n, n_batch), 1)
    avg = jnp.where(bidx == bcol, jnp.float32(1.0 / tf), 0.0)
    ta = jax.lax.dot_general(emb, avg, (((1,), (0,)), ((), ())),
                             precision=_HI,
                             preferred_element_type=jnp.float32)
    lin = jax.lax.dot_general(lw_ref[...].astype(jnp.bfloat16),
                              ta.astype(jnp.bfloat16), (((1,), (0,)), ((), ())),
                              preferred_element_type=jnp.float32) + lb_ref[...]
    gb = jax.lax.dot_general(fw_ref[...].astype(jnp.bfloat16),
                             lin.astype(jnp.bfloat16), (((1,), (0,)), ((), ())),
                             preferred_element_type=jnp.float32) + fb_ref[...]
    gamma = gb[:c, :]
    beta = gb[c:, :]
    for b in range(n_batch):
        lo, hi = b * tf, (b + 1) * tf
        o_ref[:, lo:hi] = (gamma[:, b:b + 1] * emb_r[:, lo:hi]
                           + beta[:, b:b + 1])


@jax.jit
def _vq_film(emb, cb, lw, lb, fw, fb):
    c, n = emb.shape
    n_batch, tf = 4, n // 4
    return pl.pallas_call(
        functools.partial(_vq_body, n_batch=n_batch, tf=tf),
        out_shape=jax.ShapeDtypeStruct((c, n), jnp.float32),
    )(emb, cb, lw, lb[:, None], fw, fb[:, None])


def _im2col(y, stride, pad_l, pad_r):
    c, b, t = y.shape
    to = t // stride
    ypad = jnp.pad(y, ((0, 0), (0, 0), (pad_l, pad_r)))
    taps = [ypad[:, :, k:k + stride * to:stride] for k in range(7)]
    col = jnp.stack(taps, axis=1)
    return col.reshape(c * 7, b * to)


def _shifts(x, offsets):
    c, b, t = x.shape
    outs = []
    for m in offsets:
        if m < 0:
            sh = jnp.pad(x, ((0, 0), (0, 0), (-m, 0)))[:, :, :t]
        elif m > 0:
            sh = jnp.pad(x, ((0, 0), (0, 0), (0, m)))[:, :, m:]
        else:
            sh = x
        outs.append(sh)
    return jnp.concatenate(outs, axis=0).reshape(len(offsets) * c, b * t)


def _poly_w4(w):
    o, c, _ = w.shape
    wp = jnp.zeros((4 * o, 3 * c), jnp.float32)
    place = [(0, 0, 1), (0, 1, 5), (1, 0, 0), (1, 1, 4),
             (2, 1, 3), (3, 1, 2), (3, 2, 6)]
    for p, m, k in place:
        wp = wp.at[p * o:(p + 1) * o, m * c:(m + 1) * c].set(w[:, :, k])
    return wp


def _poly_w2(w):
    o, c, _ = w.shape
    wp = jnp.zeros((2 * o, 4 * c), jnp.float32)
    place = [(0, 0, 0), (0, 1, 2), (0, 2, 4), (0, 3, 6),
             (1, 1, 1), (1, 2, 3), (1, 3, 5)]
    for p, m, k in place:
        wp = wp.at[p * o:(p + 1) * o, m * c:(m + 1) * c].set(w[:, :, k])
    return wp


def _deinterleave(z, s, o, b, t):
    return z.reshape(s, o, b, t).transpose(1, 2, 3, 0).reshape(o, b, s * t)


def kernel(wav, enc_w1, enc_b1, enc_w2, enc_b2, enc_w3, enc_b3, enc_w4,
           enc_b4, codebook, lin_w, lin_b, film_w, film_b,
           dec_w1, dec_b1, dec_w2, dec_b2, dec_w3, dec_b3, dec_w4, dec_b4):
    B = wav.shape[0]
    T = wav.shape[2]

    x0 = wav.transpose(1, 0, 2)
    col1 = _im2col(x0, 2, 2, 3)
    col1 = jnp.pad(col1, ((0, 1), (0, 0)))
    w1 = jnp.pad(enc_w1.reshape(64, 7), ((0, 0), (0, 1)))
    y1 = _mm(w1, col1, enc_b1[:, None], "lrelu", nb=4096)

    col2 = _im2col(y1.reshape(64, B, T // 2), 4, 1, 2)
    y2 = _mm(enc_w2.reshape(128, 448), col2, enc_b2[:, None], "lrelu",
             nb=2048)

    col3 = _im2col(y2.reshape(128, B, T // 8), 4, 1, 2)
    y3 = _mm(enc_w3.reshape(256, 896), col3, enc_b3[:, None], "lrelu",
             nb=1024)

    col4 = _im2col(y3.reshape(256, B, T // 32), 4, 1, 2)
    emb = _mm(enc_w4.reshape(512, 1792), col4, enc_b4[:, None], "none",
              nb=512)

    mod = _vq_film(emb, codebook, lin_w, lin_b, film_w, film_b)

    tf = T // 128
    h = mod.reshape(512, B, tf)
    z1 = _mm(_poly_w4(dec_w1), _shifts(h, (-1, 0, 1)),
             jnp.tile(dec_b1, 4)[:, None], "lrelu", nb=512)
    h = _deinterleave(z1, 4, 256, B, tf)

    z2 = _mm(_poly_w4(dec_w2), _shifts(h, (-1, 0, 1)),
             jnp.tile(dec_b2, 4)[:, None], "lrelu", nb=1024)
    h = _deinterleave(z2, 4, 128, B, 4 * tf)

    z3 = _mm(_poly_w4(dec_w3), _shifts(h, (-1, 0, 1)),
             jnp.tile(dec_b3, 4)[:, None], "lrelu", nb=2048)
    h = _deinterleave(z3, 4, 64, B, 16 * tf)

    w4p = jnp.pad(_poly_w2(dec_w4), ((0, 6), (0, 0)))
    b4p = jnp.pad(jnp.tile(dec_b4, 2), (0, 6))[:, None]
    z4 = _mm(w4p, _shifts(h, (-2, -1, 0, 1)), b4p, "tanh",
             nb=4096)
    out = _deinterleave(z4[:2], 2, 1, B, T // 2)
    return out.transpose(1, 0, 2)


# speedup vs baseline: 2.5640x; 2.5640x over previous
import functools

import jax
import jax.numpy as jnp
from jax.experimental import pallas as pl

_HI = jax.lax.Precision.HIGHEST


def _mm_body(*refs, act, n_in):
    x_refs = refs[:n_in]
    w_refs = refs[n_in:2 * n_in]
    b_ref = refs[2 * n_in]
    o_ref = refs[2 * n_in + 1]
    y = b_ref[...]
    for x_ref, w_ref in zip(x_refs, w_refs):
        y = y + jax.lax.dot_general(
            x_ref[...].astype(jnp.bfloat16), w_ref[...].astype(jnp.bfloat16),
            (((1,), (0,)), ((), ())), preferred_element_type=jnp.float32)
    if act == "lrelu":
        y = jnp.where(y > 0, y, 0.1 * y)
    elif act == "tanh":
        y = jnp.tanh(y)
    o_ref[...] = y


def _mm(xs, ws, bias, act="none", nbr=2048):
    n = xs[0].shape[0]
    o = ws[0].shape[1]
    nbr = min(nbr, n)
    in_specs = [pl.BlockSpec((nbr, x.shape[1]), lambda i: (i, 0)) for x in xs]
    in_specs += [pl.BlockSpec(w.shape, lambda i: (0, 0)) for w in ws]
    in_specs += [pl.BlockSpec((1, o), lambda i: (0, 0))]
    return pl.pallas_call(
        functools.partial(_mm_body, act=act, n_in=len(xs)),
        grid=(n // nbr,),
        in_specs=in_specs,
        out_specs=pl.BlockSpec((nbr, o), lambda i: (i, 0)),
        out_shape=jax.ShapeDtypeStruct((n, o), jnp.float32),
    )(*xs, *ws, bias)


def _vq_body(emb_ref, cb_ref, lw_ref, lb_ref, fw_ref, fb_ref, o_ref,
             *, n_batch, tf):
    emb = emb_ref[...]
    cb = cb_ref[...]
    kk, c = cb.shape
    n = emb.shape[0]
    s = jax.lax.dot_general(emb.astype(jnp.bfloat16), cb.astype(jnp.bfloat16),
                            (((1,), (1,)), ((), ())),
                            preferred_element_type=jnp.float32)
    cn2 = jax.lax.dot_general(jnp.full((1, c), 1.0, jnp.float32), cb * cb,
                              (((1,), (1,)), ((), ())),
                              precision=_HI,
                              preferred_element_type=jnp.float32)
    d = cn2 - 2.0 * s
    dmin = jnp.min(d, axis=1, keepdims=True)
    jidx = jax.lax.broadcasted_iota(jnp.int32, (n, kk), 1)
    codes = jnp.min(jnp.where(d == dmin, jidx, kk), axis=1, keepdims=True)
    onehot = (jidx == codes).astype(jnp.float32)
    emb_r = jax.lax.dot_general(onehot, cb, (((1,), (0,)), ((), ())),
                                precision=_HI,
                                preferred_element_type=jnp.float32)
    bidx = jax.lax.broadcasted_iota(jnp.int32, (n_batch, n), 1) // tf
    brow = jax.lax.broadcasted_iota(jnp.int32, (n_batch, n), 0)
    avg = jnp.where(bidx == brow, jnp.float32(1.0 / tf), 0.0)
    ta = jax.lax.dot_general(avg, emb, (((1,), (0,)), ((), ())),
                             precision=_HI,
                             preferred_element_type=jnp.float32)
    lin = jax.lax.dot_general(ta.astype(jnp.bfloat16),
                              lw_ref[...].astype(jnp.bfloat16),
                              (((1,), (1,)), ((), ())),
                              preferred_element_type=jnp.float32) + lb_ref[...]
    gb = jax.lax.dot_general(lin.astype(jnp.bfloat16),
                             fw_ref[...].astype(jnp.bfloat16),
                             (((1,), (1,)), ((), ())),
                             preferred_element_type=jnp.float32) + fb_ref[...]
    gamma = gb[:, :c]
    beta = gb[:, c:]
    for b in range(n_batch):
        lo, hi = b * tf, (b + 1) * tf
        o_ref[lo:hi, :] = (emb_r[lo:hi, :] * gamma[b:b + 1, :]
                           + beta[b:b + 1, :])


def _vq_film(emb, cb, lw, lb, fw, fb):
    n, c = emb.shape
    n_batch, tf = 4, n // 4
    return pl.pallas_call(
        functools.partial(_vq_body, n_batch=n_batch, tf=tf),
        out_shape=jax.ShapeDtypeStruct((n, c), jnp.float32),
    )(emb, cb, lw, lb[None, :], fw, fb[None, :])


def _frame_shifts(x, offsets):
    b, f, dd = x.shape
    outs = []
    for m in offsets:
        if m < 0:
            sh = jnp.pad(x, ((0, 0), (-m, 0), (0, 0)))[:, :f]
        elif m > 0:
            sh = jnp.pad(x, ((0, 0), (0, m), (0, 0)))[:, m:]
        else:
            sh = x
        outs.append(sh.reshape(b * f, dd))
    return outs


def _enc_w(w, s, pad_l, offsets):
    o, c, _ = w.shape
    mats = []
    for d in offsets:
        m = jnp.zeros((s * c, o), jnp.float32)
        for j in range(s):
            k = s * d + j + pad_l
            if 0 <= k < 7:
                m = m.at[j * c:(j + 1) * c, :].set(w[:, :, k].T)
        mats.append(m)
    return mats


def _dec_w(w, s, pad_a, offsets):
    o, c, _ = w.shape
    mats = []
    for m in offsets:
        mat = jnp.zeros((c, s * o), jnp.float32)
        for p in range(s):
            k = s * m + pad_a - p
            if 0 <= k < 7:
                mat = mat.at[:, p * o:(p + 1) * o].set(w[:, :, k].T)
        mats.append(mat)
    return mats


def kernel(wav, enc_w1, enc_b1, enc_w2, enc_b2, enc_w3, enc_b3, enc_w4,
           enc_b4, codebook, lin_w, lin_b, film_w, film_b,
           dec_w1, dec_b1, dec_w2, dec_b2, dec_w3, dec_b3, dec_w4, dec_b4):
    B = wav.shape[0]
    T = wav.shape[2]

    x = wav.reshape(B, T // 2, 2)
    y = _mm(_frame_shifts(x, (-1, 0, 1, 2)),
            _enc_w(enc_w1, 2, 2, (-1, 0, 1, 2)),
            enc_b1[None, :], "lrelu", nbr=4096)

    x = y.reshape(B, T // 8, 4 * 64)
    y = _mm(_frame_shifts(x, (-1, 0, 1)),
            _enc_w(enc_w2, 4, 1, (-1, 0, 1)),
            enc_b2[None, :], "lrelu", nbr=2048)

    x = y.reshape(B, T // 32, 4 * 128)
    y = _mm(_frame_shifts(x, (-1, 0, 1)),
            _enc_w(enc_w3, 4, 1, (-1, 0, 1)),
            enc_b3[None, :], "lrelu", nbr=1024)

    x = y.reshape(B, T // 128, 4 * 256)
    emb = _mm(_frame_shifts(x, (-1, 0, 1)),
              _enc_w(enc_w4, 4, 1, (-1, 0, 1)),
              enc_b4[None, :], "none", nbr=1024)

    mod = _vq_film(emb, codebook, lin_w, lin_b, film_w, film_b)

    tf = T // 128
    x = mod.reshape(B, tf, 512)
    z = _mm(_frame_shifts(x, (-1, 0, 1)),
            _dec_w(dec_w1, 4, 5, (-1, 0, 1)),
            jnp.tile(dec_b1, 4)[None, :], "lrelu", nbr=1024)

    x = z.reshape(B, 4 * tf, 256)
    z = _mm(_frame_shifts(x, (-1, 0, 1)),
            _dec_w(dec_w2, 4, 5, (-1, 0, 1)),
            jnp.tile(dec_b2, 4)[None, :], "lrelu", nbr=1024)

    x = z.reshape(B, 16 * tf, 128)
    z = _mm(_frame_shifts(x, (-1, 0, 1)),
            _dec_w(dec_w3, 4, 5, (-1, 0, 1)),
            jnp.tile(dec_b3, 4)[None, :], "lrelu", nbr=2048)

    x = z.reshape(B, 64 * tf, 64)
    z = _mm(_frame_shifts(x, (-2, -1, 0, 1)),
            _dec_w(dec_w4, 2, 4, (-2, -1, 0, 1)),
            jnp.tile(dec_b4, 2)[None, :], "tanh", nbr=4096)

    return z.reshape(B, 1, T)


# speedup vs baseline: 5.8324x; 2.2747x over previous
import functools

import jax
import jax.numpy as jnp
import numpy as np
from jax.experimental import pallas as pl

_HI = jax.lax.Precision.HIGHEST


def _mm_body(*refs, act, n_in):
    x_refs = refs[:n_in]
    w_refs = refs[n_in:2 * n_in]
    b_ref = refs[2 * n_in]
    o_ref = refs[2 * n_in + 1]
    y = b_ref[...]
    for x_ref, w_ref in zip(x_refs, w_refs):
        y = y + jax.lax.dot_general(
            x_ref[...].astype(jnp.bfloat16), w_ref[...].astype(jnp.bfloat16),
            (((1,), (0,)), ((), ())), preferred_element_type=jnp.float32)
    if act == "lrelu":
        y = jnp.where(y > 0, y, 0.1 * y)
    elif act == "tanh":
        y = jnp.tanh(y)
    o_ref[...] = y


def _mm(xs, ws, bias, act="none", nbr=2048):
    n = xs[0].shape[0]
    o = ws[0].shape[1]
    nbr = min(nbr, n)
    in_specs = [pl.BlockSpec((nbr, x.shape[1]), lambda i: (i, 0)) for x in xs]
    in_specs += [pl.BlockSpec(w.shape, lambda i: (0, 0)) for w in ws]
    in_specs += [pl.BlockSpec((1, o), lambda i: (0, 0))]
    return pl.pallas_call(
        functools.partial(_mm_body, act=act, n_in=len(xs)),
        grid=(n // nbr,),
        in_specs=in_specs,
        out_specs=pl.BlockSpec((nbr, o), lambda i: (i, 0)),
        out_shape=jax.ShapeDtypeStruct((n, o), jnp.float32),
    )(*xs, *ws, bias)


def _vq_body(emb_ref, cb_ref, lw_ref, lb_ref, fw_ref, fb_ref, o_ref,
             *, n_batch, tf):
    emb = emb_ref[...]
    cb = cb_ref[...]
    kk, c = cb.shape
    n = emb.shape[0]
    s = jax.lax.dot_general(emb.astype(jnp.bfloat16), cb.astype(jnp.bfloat16),
                            (((1,), (1,)), ((), ())),
                            preferred_element_type=jnp.float32)
    cn2 = jax.lax.dot_general(jnp.full((1, c), 1.0, jnp.float32), cb * cb,
                              (((1,), (1,)), ((), ())),
                              precision=_HI,
                              preferred_element_type=jnp.float32)
    d = cn2 - 2.0 * s
    dmin = jnp.min(d, axis=1, keepdims=True)
    jidx = jax.lax.broadcasted_iota(jnp.int32, (n, kk), 1)
    codes = jnp.min(jnp.where(d == dmin, jidx, kk), axis=1, keepdims=True)
    onehot = (jidx == codes).astype(jnp.float32)
    emb_r = jax.lax.dot_general(onehot, cb, (((1,), (0,)), ((), ())),
                                precision=_HI,
                                preferred_element_type=jnp.float32)
    bidx = jax.lax.broadcasted_iota(jnp.int32, (n_batch, n), 1) // tf
    brow = jax.lax.broadcasted_iota(jnp.int32, (n_batch, n), 0)
    avg = jnp.where(bidx == brow, jnp.float32(1.0 / tf), 0.0)
    ta = jax.lax.dot_general(avg, emb, (((1,), (0,)), ((), ())),
                             precision=_HI,
                             preferred_element_type=jnp.float32)
    lin = jax.lax.dot_general(ta.astype(jnp.bfloat16),
                              lw_ref[...].astype(jnp.bfloat16),
                              (((1,), (1,)), ((), ())),
                              preferred_element_type=jnp.float32) + lb_ref[...]
    gb = jax.lax.dot_general(lin.astype(jnp.bfloat16),
                             fw_ref[...].astype(jnp.bfloat16),
                             (((1,), (1,)), ((), ())),
                             preferred_element_type=jnp.float32) + fb_ref[...]
    gamma = gb[:, :c]
    beta = gb[:, c:]
    for b in range(n_batch):
        lo, hi = b * tf, (b + 1) * tf
        o_ref[lo:hi, :] = (emb_r[lo:hi, :] * gamma[b:b + 1, :]
                           + beta[b:b + 1, :])


def _vq_film(emb, cb, lw, lb, fw, fb):
    n, c = emb.shape
    n_batch, tf = 4, n // 4
    return pl.pallas_call(
        functools.partial(_vq_body, n_batch=n_batch, tf=tf),
        out_shape=jax.ShapeDtypeStruct((n, c), jnp.float32),
    )(emb, cb, lw, lb[None, :], fw, fb[None, :])


def _frame_shifts(x, offsets):
    b, f, dd = x.shape
    outs = []
    for m in offsets:
        if m < 0:
            sh = jnp.pad(x, ((0, 0), (-m, 0), (0, 0)))[:, :f]
        elif m > 0:
            sh = jnp.pad(x, ((0, 0), (0, m), (0, 0)))[:, m:]
        else:
            sh = x
        outs.append(sh.reshape(b * f, dd))
    return outs


def _enc_w(w, s, pad_l, offsets):
    o, c, _ = w.shape
    mats = []
    for d in offsets:
        m = jnp.zeros((s * c, o), jnp.float32)
        for j in range(s):
            k = s * d + j + pad_l
            if 0 <= k < 7:
                m = m.at[j * c:(j + 1) * c, :].set(w[:, :, k].T)
        mats.append(m)
    return mats


def _dec_w(w, s, pad_a, offsets):
    o, c, _ = w.shape
    mats = []
    for m in offsets:
        mat = jnp.zeros((c, s * o), jnp.float32)
        for p in range(s):
            k = s * m + pad_a - p
            if 0 <= k < 7:
                mat = mat.at[:, p * o:(p + 1) * o].set(w[:, :, k].T)
        mats.append(mat)
    return mats


def _e1_w(w1):
    mats = []
    for dshift in (-1, 0, 1):
        m = np.zeros((7, 128, 16, 4), np.float32)
        for a in range(16):
            for j in range(4):
                for k in range(7):
                    q = 8 * a + 2 * j + k - 2 - 128 * dshift
                    if 0 <= q < 128:
                        m[k, q, a, j] = 1.0
        mats.append(jnp.einsum('kqaj,ck->qajc', jnp.asarray(m),
                               w1[:, 0, :]).reshape(128, 4096))
    return mats


def _d4_w(w4):
    mats = []
    for dshift in (-1, 0, 1):
        m = np.zeros((7, 64, 64, 2), np.float32)
        for q in range(64):
            for j in range(64):
                for p in range(2):
                    k = 2 * (q + 64 * dshift - j) + 4 - p
                    if 0 <= k < 7:
                        m[k, q, j, p] = 1.0
        mats.append(jnp.einsum('kqjp,ck->qcjp', jnp.asarray(m),
                               w4[0]).reshape(4096, 128))
    return mats


def kernel(wav, enc_w1, enc_b1, enc_w2, enc_b2, enc_w3, enc_b3, enc_w4,
           enc_b4, codebook, lin_w, lin_b, film_w, film_b,
           dec_w1, dec_b1, dec_w2, dec_b2, dec_w3, dec_b3, dec_w4, dec_b4):
    B = wav.shape[0]
    T = wav.shape[2]

    x = wav.reshape(B, T // 128, 128)
    y = _mm(_frame_shifts(x, (-1, 0, 1)), _e1_w(enc_w1),
            jnp.tile(enc_b1, 64)[None, :], "lrelu", nbr=1024)

    x = y.reshape(B, T // 8, 4 * 64)
    y = _mm(_frame_shifts(x, (-1, 0, 1)),
            _enc_w(enc_w2, 4, 1, (-1, 0, 1)),
            enc_b2[None, :], "lrelu", nbr=2048)

    x = y.reshape(B, T // 32, 4 * 128)
    y = _mm(_frame_shifts(x, (-1, 0, 1)),
            _enc_w(enc_w3, 4, 1, (-1, 0, 1)),
            enc_b3[None, :], "lrelu", nbr=1024)

    x = y.reshape(B, T // 128, 4 * 256)
    emb = _mm(_frame_shifts(x, (-1, 0, 1)),
              _enc_w(enc_w4, 4, 1, (-1, 0, 1)),
              enc_b4[None, :], "none", nbr=1024)

    mod = _vq_film(emb, codebook, lin_w, lin_b, film_w, film_b)

    tf = T // 128
    x = mod.reshape(B, tf, 512)
    z = _mm(_frame_shifts(x, (-1, 0, 1)),
            _dec_w(dec_w1, 4, 5, (-1, 0, 1)),
            jnp.tile(dec_b1, 4)[None, :], "lrelu", nbr=1024)

    x = z.reshape(B, 4 * tf, 256)
    z = _mm(_frame_shifts(x, (-1, 0, 1)),
            _dec_w(dec_w2, 4, 5, (-1, 0, 1)),
            jnp.tile(dec_b2, 4)[None, :], "lrelu", nbr=1024)

    x = z.reshape(B, 16 * tf, 128)
    z = _mm(_frame_shifts(x, (-1, 0, 1)),
            _dec_w(dec_w3, 4, 5, (-1, 0, 1)),
            jnp.tile(dec_b3, 4)[None, :], "lrelu", nbr=2048)

    x = z.reshape(B, T // 128, 64 * 64)
    z = _mm(_frame_shifts(x, (-1, 0, 1)), _d4_w(dec_w4),
            jnp.broadcast_to(dec_b4, (1, 128)), "tanh",
            nbr=512)

    return z.reshape(B, 1, T)


# speedup vs baseline: 9.0577x; 1.5530x over previous
import functools

import jax
import jax.numpy as jnp
import numpy as np
from jax.experimental import pallas as pl

_HI = jax.lax.Precision.HIGHEST


def _mm_body(*refs, act, n_in):
    x_refs = refs[:n_in]
    w_refs = refs[n_in:2 * n_in]
    b_ref = refs[2 * n_in]
    o_ref = refs[2 * n_in + 1]
    y = b_ref[...]
    for x_ref, w_ref in zip(x_refs, w_refs):
        y = y + jax.lax.dot_general(
            x_ref[...].astype(jnp.bfloat16), w_ref[...].astype(jnp.bfloat16),
            (((1,), (0,)), ((), ())), preferred_element_type=jnp.float32)
    if act == "lrelu":
        y = jnp.where(y > 0, y, 0.1 * y)
    elif act == "tanh":
        y = jnp.tanh(y)
    o_ref[...] = y


def _mm(xs, ws, bias, act="none", nbr=2048):
    n = xs[0].shape[0]
    o = ws[0].shape[1]
    nbr = min(nbr, n)
    in_specs = [pl.BlockSpec((nbr, x.shape[1]), lambda i: (i, 0)) for x in xs]
    in_specs += [pl.BlockSpec(w.shape, lambda i: (0, 0)) for w in ws]
    in_specs += [pl.BlockSpec((1, o), lambda i: (0, 0))]
    return pl.pallas_call(
        functools.partial(_mm_body, act=act, n_in=len(xs)),
        grid=(n // nbr,),
        in_specs=in_specs,
        out_specs=pl.BlockSpec((nbr, o), lambda i: (i, 0)),
        out_shape=jax.ShapeDtypeStruct((n, o), jnp.float32),
    )(*xs, *ws, bias)


def _conv_body(xp_ref, xc_ref, xn_ref, wm_ref, w0_ref, wp_ref, b_ref, o_ref,
               *, act, nbf, period):
    i = pl.program_id(0)
    cur = xc_ref[...]
    first = (jax.lax.rem(i, period) == 0).astype(jnp.float32)
    last = (jax.lax.rem(i, period) == period - 1).astype(jnp.float32)
    prev_row = xp_ref[nbf - 1:nbf, :] * (1.0 - first)
    next_row = xn_ref[0:1, :] * (1.0 - last)
    xm = jnp.concatenate([prev_row, cur[:nbf - 1, :]], axis=0)
    xn = jnp.concatenate([cur[1:, :], next_row], axis=0)
    y = b_ref[...]
    for xx, w_ref in ((xm, wm_ref), (cur, w0_ref), (xn, wp_ref)):
        y = y + jax.lax.dot_general(
            xx.astype(jnp.bfloat16), w_ref[...].astype(jnp.bfloat16),
            (((1,), (0,)), ((), ())), preferred_element_type=jnp.float32)
    if act == "lrelu":
        y = jnp.where(y > 0, y, 0.1 * y)
    elif act == "tanh":
        y = jnp.tanh(y)
    o_ref[...] = y


def _conv_mm(x3, ws, bias, act="none", nbf=2048):
    b, f, dd = x3.shape
    nbf = min(nbf, f)
    assert f % nbf == 0
    period = f // nbf
    n = b * f
    o = ws[0].shape[1]
    x = x3.reshape(n, dd)
    g = n // nbf
    specs = [
        pl.BlockSpec((nbf, dd), lambda i: (jnp.maximum(i - 1, 0), 0)),
        pl.BlockSpec((nbf, dd), lambda i: (i, 0)),
        pl.BlockSpec((nbf, dd), lambda i: (jnp.minimum(i + 1, g - 1), 0)),
    ]
    specs += [pl.BlockSpec(w.shape, lambda i: (0, 0)) for w in ws]
    specs += [pl.BlockSpec((1, o), lambda i: (0, 0))]
    return pl.pallas_call(
        functools.partial(_conv_body, act=act, nbf=nbf, period=period),
        grid=(g,),
        in_specs=specs,
        out_specs=pl.BlockSpec((nbf, o), lambda i: (i, 0)),
        out_shape=jax.ShapeDtypeStruct((n, o), jnp.float32),
    )(x, x, x, *ws, bias)


def _vq_body(emb_ref, cb_ref, lw_ref, lb_ref, fw_ref, fb_ref, o_ref,
             *, n_batch, tf):
    emb = emb_ref[...]
    cb = cb_ref[...]
    kk, c = cb.shape
    n = emb.shape[0]
    s = jax.lax.dot_general(emb.astype(jnp.bfloat16), cb.astype(jnp.bfloat16),
                            (((1,), (1,)), ((), ())),
                            preferred_element_type=jnp.float32)
    cn2 = jax.lax.dot_general(jnp.full((1, c), 1.0, jnp.float32), cb * cb,
                              (((1,), (1,)), ((), ())),
                              precision=_HI,
                              preferred_element_type=jnp.float32)
    d = cn2 - 2.0 * s
    dmin = jnp.min(d, axis=1, keepdims=True)
    jidx = jax.lax.broadcasted_iota(jnp.int32, (n, kk), 1)
    codes = jnp.min(jnp.where(d == dmin, jidx, kk), axis=1, keepdims=True)
    onehot = (jidx == codes).astype(jnp.float32)
    emb_r = jax.lax.dot_general(onehot, cb, (((1,), (0,)), ((), ())),
                                precision=_HI,
                                preferred_element_type=jnp.float32)
    bidx = jax.lax.broadcasted_iota(jnp.int32, (n_batch, n), 1) // tf
    brow = jax.lax.broadcasted_iota(jnp.int32, (n_batch, n), 0)
    avg = jnp.where(bidx == brow, jnp.float32(1.0 / tf), 0.0)
    ta = jax.lax.dot_general(avg, emb, (((1,), (0,)), ((), ())),
                             precision=_HI,
                             preferred_element_type=jnp.float32)
    lin = jax.lax.dot_general(ta.astype(jnp.bfloat16),
                              lw_ref[...].astype(jnp.bfloat16),
                              (((1,), (1,)), ((), ())),
                              preferred_element_type=jnp.float32) + lb_ref[...]
    gb = jax.lax.dot_general(lin.astype(jnp.bfloat16),
                             fw_ref[...].astype(jnp.bfloat16),
                             (((1,), (1,)), ((), ())),
                             preferred_element_type=jnp.float32) + fb_ref[...]
    gamma = gb[:, :c]
    beta = gb[:, c:]
    for b in range(n_batch):
        lo, hi = b * tf, (b + 1) * tf
        o_ref[lo:hi, :] = (emb_r[lo:hi, :] * gamma[b:b + 1, :]
                           + beta[b:b + 1, :])


def _vq_film(emb, cb, lw, lb, fw, fb):
    n, c = emb.shape
    n_batch, tf = 4, n // 4
    return pl.pallas_call(
        functools.partial(_vq_body, n_batch=n_batch, tf=tf),
        out_shape=jax.ShapeDtypeStruct((n, c), jnp.float32),
    )(emb, cb, lw, lb[None, :], fw, fb[None, :])


def _frame_shifts(x, offsets):
    b, f, dd = x.shape
    outs = []
    for m in offsets:
        if m < 0:
            sh = jnp.pad(x, ((0, 0), (-m, 0), (0, 0)))[:, :f]
        elif m > 0:
            sh = jnp.pad(x, ((0, 0), (0, m), (0, 0)))[:, m:]
        else:
            sh = x
        outs.append(sh.reshape(b * f, dd))
    return outs


def _enc_w(w, s, pad_l, offsets):
    o, c, _ = w.shape
    mats = []
    for d in offsets:
        m = jnp.zeros((s * c, o), jnp.float32)
        for j in range(s):
            k = s * d + j + pad_l
            if 0 <= k < 7:
                m = m.at[j * c:(j + 1) * c, :].set(w[:, :, k].T)
        mats.append(m)
    return mats


def _dec_w(w, s, pad_a, offsets):
    o, c, _ = w.shape
    mats = []
    for m in offsets:
        mat = jnp.zeros((c, s * o), jnp.float32)
        for p in range(s):
            k = s * m + pad_a - p
            if 0 <= k < 7:
                mat = mat.at[:, p * o:(p + 1) * o].set(w[:, :, k].T)
        mats.append(mat)
    return mats


def _e1_w(w1):
    mats = []
    for dshift in (-1, 0, 1):
        m = np.zeros((7, 128, 16, 4), np.float32)
        for a in range(16):
            for j in range(4):
                for k in range(7):
                    q = 8 * a + 2 * j + k - 2 - 128 * dshift
                    if 0 <= q < 128:
                        m[k, q, a, j] = 1.0
        mats.append(jnp.einsum('kqaj,ck->qajc', jnp.asarray(m),
                               w1[:, 0, :]).reshape(128, 4096))
    return mats


def _d4_w(w4):
    mats = []
    for dshift in (-1, 0, 1):
        m = np.zeros((7, 64, 64, 2), np.float32)
        for q in range(64):
            for j in range(64):
                for p in range(2):
                    k = 2 * (q + 64 * dshift - j) + 4 - p
                    if 0 <= k < 7:
                        m[k, q, j, p] = 1.0
        mats.append(jnp.einsum('kqjp,ck->qcjp', jnp.asarray(m),
                               w4[0]).reshape(4096, 128))
    return mats


def kernel(wav, enc_w1, enc_b1, enc_w2, enc_b2, enc_w3, enc_b3, enc_w4,
           enc_b4, codebook, lin_w, lin_b, film_w, film_b,
           dec_w1, dec_b1, dec_w2, dec_b2, dec_w3, dec_b3, dec_w4, dec_b4):
    B = wav.shape[0]
    T = wav.shape[2]

    x = wav.reshape(B, T // 128, 128)
    y = _conv_mm(x, _e1_w(enc_w1),
                 jnp.tile(enc_b1, 64)[None, :], "lrelu", nbf=256)

    x = y.reshape(B, T // 8, 4 * 64)
    y = _conv_mm(x, _enc_w(enc_w2, 4, 1, (-1, 0, 1)),
                 enc_b2[None, :], "lrelu", nbf=2048)

    x = y.reshape(B, T // 32, 4 * 128)
    y = _conv_mm(x, _enc_w(enc_w3, 4, 1, (-1, 0, 1)),
                 enc_b3[None, :], "lrelu", nbf=1024)

    x = y.reshape(B, T // 128, 4 * 256)
    emb = _conv_mm(x, _enc_w(enc_w4, 4, 1, (-1, 0, 1)),
                   enc_b4[None, :], "none", nbf=256)

    mod = _vq_film(emb, codebook, lin_w, lin_b, film_w, film_b)

    tf = T // 128
    x = mod.reshape(B, tf, 512)
    z = _conv_mm(x, _dec_w(dec_w1, 4, 5, (-1, 0, 1)),
                 jnp.tile(dec_b1, 4)[None, :], "lrelu", nbf=256)

    x = z.reshape(B, 4 * tf, 256)
    z = _conv_mm(x, _dec_w(dec_w2, 4, 5, (-1, 0, 1)),
                 jnp.tile(dec_b2, 4)[None, :], "lrelu", nbf=1024)

    x = z.reshape(B, 16 * tf, 128)
    z = _conv_mm(x, _dec_w(dec_w3, 4, 5, (-1, 0, 1)),
                 jnp.tile(dec_b3, 4)[None, :], "lrelu", nbf=2048)

    x = z.reshape(B, T // 128, 64 * 64)
    z = _conv_mm(x, _d4_w(dec_w4),
                 jnp.broadcast_to(dec_b4, (1, 128)), "tanh",
                 nbf=256)

    return z.reshape(B, 1, T)


# speedup vs baseline: 10.6180x; 1.1723x over previous
import functools

import jax
import jax.numpy as jnp
import numpy as np
from jax.experimental import pallas as pl

_HI = jax.lax.Precision.HIGHEST


def _mm_body(*refs, act, n_in):
    x_refs = refs[:n_in]
    w_refs = refs[n_in:2 * n_in]
    b_ref = refs[2 * n_in]
    o_ref = refs[2 * n_in + 1]
    y = b_ref[...]
    for x_ref, w_ref in zip(x_refs, w_refs):
        y = y + jax.lax.dot_general(
            x_ref[...].astype(jnp.bfloat16), w_ref[...].astype(jnp.bfloat16),
            (((1,), (0,)), ((), ())), preferred_element_type=jnp.float32)
    if act == "lrelu":
        y = jnp.where(y > 0, y, 0.1 * y)
    elif act == "tanh":
        y = jnp.tanh(y)
    o_ref[...] = y


def _mm(xs, ws, bias, act="none", nbr=2048):
    n = xs[0].shape[0]
    o = ws[0].shape[1]
    nbr = min(nbr, n)
    in_specs = [pl.BlockSpec((nbr, x.shape[1]), lambda i: (i, 0)) for x in xs]
    in_specs += [pl.BlockSpec(w.shape, lambda i: (0, 0)) for w in ws]
    in_specs += [pl.BlockSpec((1, o), lambda i: (0, 0))]
    return pl.pallas_call(
        functools.partial(_mm_body, act=act, n_in=len(xs)),
        grid=(n // nbr,),
        in_specs=in_specs,
        out_specs=pl.BlockSpec((nbr, o), lambda i: (i, 0)),
        out_shape=jax.ShapeDtypeStruct((n, o), jnp.float32),
    )(*xs, *ws, bias)


def _conv_body(xp_ref, xc_ref, xn_ref, wm_ref, w0_ref, wp_ref, b_ref, o_ref,
               *, act, nbf, period):
    i = pl.program_id(0)
    cur = xc_ref[...]
    pr = xp_ref[nbf - 1:nbf, :]
    nx = xn_ref[0:1, :]
    prev_row = jnp.where(jax.lax.rem(i, period) == 0, jnp.zeros_like(pr), pr)
    next_row = jnp.where(jax.lax.rem(i, period) == period - 1,
                         jnp.zeros_like(nx), nx)
    xm = jnp.concatenate([prev_row, cur[:nbf - 1, :]], axis=0)
    xn = jnp.concatenate([cur[1:, :], next_row], axis=0)
    y = b_ref[...]
    for xx, w_ref in ((xm, wm_ref), (cur, w0_ref), (xn, wp_ref)):
        y = y + jax.lax.dot_general(
            xx.astype(jnp.bfloat16), w_ref[...].astype(jnp.bfloat16),
            (((1,), (0,)), ((), ())), preferred_element_type=jnp.float32)
    if act == "lrelu":
        y = jnp.where(y > 0, y, 0.1 * y)
    elif act == "tanh":
        y = jnp.tanh(y)
    o_ref[...] = y.astype(o_ref.dtype)


def _conv_mm(x3, ws, bias, act="none", nbf=2048, out_dtype=jnp.bfloat16):
    b, f, dd = x3.shape
    nbf = min(nbf, f)
    assert f % nbf == 0
    period = f // nbf
    n = b * f
    o = ws[0].shape[1]
    x = x3.reshape(n, dd)
    g = n // nbf
    specs = [
        pl.BlockSpec((nbf, dd), lambda i: (jnp.maximum(i - 1, 0), 0)),
        pl.BlockSpec((nbf, dd), lambda i: (i, 0)),
        pl.BlockSpec((nbf, dd), lambda i: (jnp.minimum(i + 1, g - 1), 0)),
    ]
    specs += [pl.BlockSpec(w.shape, lambda i: (0, 0)) for w in ws]
    specs += [pl.BlockSpec((1, o), lambda i: (0, 0))]
    return pl.pallas_call(
        functools.partial(_conv_body, act=act, nbf=nbf, period=period),
        grid=(g,),
        in_specs=specs,
        out_specs=pl.BlockSpec((nbf, o), lambda i: (i, 0)),
        out_shape=jax.ShapeDtypeStruct((n, o), out_dtype),
    )(x, x, x, *ws, bias)


def _vq_body(emb_ref, cb_ref, lw_ref, lb_ref, fw_ref, fb_ref, o_ref,
             *, n_batch, tf):
    emb = emb_ref[...]
    cb = cb_ref[...]
    kk, c = cb.shape
    n = emb.shape[0]
    s = jax.lax.dot_general(emb.astype(jnp.bfloat16), cb.astype(jnp.bfloat16),
                            (((1,), (1,)), ((), ())),
                            preferred_element_type=jnp.float32)
    cn2 = jax.lax.dot_general(jnp.full((1, c), 1.0, jnp.float32), cb * cb,
                              (((1,), (1,)), ((), ())),
                              precision=_HI,
                              preferred_element_type=jnp.float32)
    d = cn2 - 2.0 * s
    dmin = jnp.min(d, axis=1, keepdims=True)
    jidx = jax.lax.broadcasted_iota(jnp.int32, (n, kk), 1)
    codes = jnp.min(jnp.where(d == dmin, jidx, kk), axis=1, keepdims=True)
    onehot = (jidx == codes).astype(jnp.float32)
    emb_r = jax.lax.dot_general(onehot, cb, (((1,), (0,)), ((), ())),
                                precision=_HI,
                                preferred_element_type=jnp.float32)
    bidx = jax.lax.broadcasted_iota(jnp.int32, (n_batch, n), 1) // tf
    brow = jax.lax.broadcasted_iota(jnp.int32, (n_batch, n), 0)
    avg = jnp.where(bidx == brow, jnp.float32(1.0 / tf), 0.0)
    ta = jax.lax.dot_general(avg, emb, (((1,), (0,)), ((), ())),
                             precision=_HI,
                             preferred_element_type=jnp.float32)
    lin = jax.lax.dot_general(ta.astype(jnp.bfloat16),
                              lw_ref[...].astype(jnp.bfloat16),
                              (((1,), (1,)), ((), ())),
                              preferred_element_type=jnp.float32) + lb_ref[...]
    gb = jax.lax.dot_general(lin.astype(jnp.bfloat16),
                             fw_ref[...].astype(jnp.bfloat16),
                             (((1,), (1,)), ((), ())),
                             preferred_element_type=jnp.float32) + fb_ref[...]
    gamma = gb[:, :c]
    beta = gb[:, c:]
    for b in range(n_batch):
        lo, hi = b * tf, (b + 1) * tf
        o_ref[lo:hi, :] = (emb_r[lo:hi, :] * gamma[b:b + 1, :]
                           + beta[b:b + 1, :]).astype(o_ref.dtype)


def _vq_film(emb, cb, lw, lb, fw, fb):
    n, c = emb.shape
    n_batch, tf = 4, n // 4
    return pl.pallas_call(
        functools.partial(_vq_body, n_batch=n_batch, tf=tf),
        out_shape=jax.ShapeDtypeStruct((n, c), jnp.bfloat16),
    )(emb, cb, lw, lb[None, :], fw, fb[None, :])


def _frame_shifts(x, offsets):
    b, f, dd = x.shape
    outs = []
    for m in offsets:
        if m < 0:
            sh = jnp.pad(x, ((0, 0), (-m, 0), (0, 0)))[:, :f]
        elif m > 0:
            sh = jnp.pad(x, ((0, 0), (0, m), (0, 0)))[:, m:]
        else:
            sh = x
        outs.append(sh.reshape(b * f, dd))
    return outs


def _enc_w(w, s, pad_l, offsets):
    o, c, _ = w.shape
    mats = []
    for d in offsets:
        m = jnp.zeros((s * c, o), jnp.float32)
        for j in range(s):
            k = s * d + j + pad_l
            if 0 <= k < 7:
                m = m.at[j * c:(j + 1) * c, :].set(w[:, :, k].T)
        mats.append(m)
    return mats


def _dec_w(w, s, pad_a, offsets):
    o, c, _ = w.shape
    mats = []
    for m in offsets:
        mat = jnp.zeros((c, s * o), jnp.float32)
        for p in range(s):
            k = s * m + pad_a - p
            if 0 <= k < 7:
                mat = mat.at[:, p * o:(p + 1) * o].set(w[:, :, k].T)
        mats.append(mat)
    return mats


def _e1_w(w1):
    mats = []
    for dshift in (-1, 0, 1):
        m = np.zeros((7, 128, 16, 4), np.float32)
        for a in range(16):
            for j in range(4):
                for k in range(7):
                    q = 8 * a + 2 * j + k - 2 - 128 * dshift
                    if 0 <= q < 128:
                        m[k, q, a, j] = 1.0
        mats.append(jnp.einsum('kqaj,ck->qajc', jnp.asarray(m),
                               w1[:, 0, :]).reshape(128, 4096))
    return mats


def _d4_w(w4):
    mats = []
    for dshift in (-1, 0, 1):
        m = np.zeros((7, 64, 64, 2), np.float32)
        for q in range(64):
            for j in range(64):
                for p in range(2):
                    k = 2 * (q + 64 * dshift - j) + 4 - p
                    if 0 <= k < 7:
                        m[k, q, j, p] = 1.0
        mats.append(jnp.einsum('kqjp,ck->qcjp', jnp.asarray(m),
                               w4[0]).reshape(4096, 128))
    return mats


def kernel(wav, enc_w1, enc_b1, enc_w2, enc_b2, enc_w3, enc_b3, enc_w4,
           enc_b4, codebook, lin_w, lin_b, film_w, film_b,
           dec_w1, dec_b1, dec_w2, dec_b2, dec_w3, dec_b3, dec_w4, dec_b4):
    B = wav.shape[0]
    T = wav.shape[2]

    x = wav.reshape(B, T // 128, 128)
    y = _conv_mm(x, _e1_w(enc_w1),
                 jnp.tile(enc_b1, 64)[None, :], "lrelu", nbf=256)

    x = y.reshape(B, T // 8, 4 * 64)
    y = _conv_mm(x, _enc_w(enc_w2, 4, 1, (-1, 0, 1)),
                 enc_b2[None, :], "lrelu", nbf=2048)

    x = y.reshape(B, T // 32, 4 * 128)
    y = _conv_mm(x, _enc_w(enc_w3, 4, 1, (-1, 0, 1)),
                 enc_b3[None, :], "lrelu", nbf=1024)

    x = y.reshape(B, T // 128, 4 * 256)
    emb = _conv_mm(x, _enc_w(enc_w4, 4, 1, (-1, 0, 1)),
                   enc_b4[None, :], "none", nbf=256,
                   out_dtype=jnp.float32)

    mod = _vq_film(emb, codebook, lin_w, lin_b, film_w, film_b)

    tf = T // 128
    x = mod.reshape(B, tf, 512)
    z = _conv_mm(x, _dec_w(dec_w1, 4, 5, (-1, 0, 1)),
                 jnp.tile(dec_b1, 4)[None, :], "lrelu", nbf=256)

    x = z.reshape(B, 4 * tf, 256)
    z = _conv_mm(x, _dec_w(dec_w2, 4, 5, (-1, 0, 1)),
                 jnp.tile(dec_b2, 4)[None, :], "lrelu", nbf=1024)

    x = z.reshape(B, 16 * tf, 128)
    z = _conv_mm(x, _dec_w(dec_w3, 4, 5, (-1, 0, 1)),
                 jnp.tile(dec_b3, 4)[None, :], "lrelu", nbf=2048)

    x = z.reshape(B, T // 128, 64 * 64)
    z = _conv_mm(x, _d4_w(dec_w4),
                 jnp.broadcast_to(dec_b4, (1, 128)), "tanh",
                 nbf=256, out_dtype=jnp.float32)

    return z.reshape(B, 1, T)
